# baseline (speedup 1.0000x reference)
import dataclasses
import functools

import jax
import jax.numpy as jnp
from jax.experimental import pallas as pl
from jax.experimental.pallas import tpu as pltpu
from jax.experimental.pallas import tpu_sc as plsc

_EPS = 0.1
_NUM_GRAPHS = 64
_LANE = 16
_NSC = 32
_CHUNK = 3136
_SENTINEL = 64


def _sc_body(
    pred_hbm, targ_hbm, batch_hbm, out_hbm, p_v, t_v, b_v, bins_v, sem_p, sem_t, sem_b
):
    cid = jax.lax.axis_index("c")
    sid = jax.lax.axis_index("s")
    chunk = cid * 16 + sid
    off = chunk * _CHUNK
    cp_p = pltpu.async_copy(pred_hbm.at[pl.ds(off, _CHUNK)], p_v, sem_p)
    cp_t = pltpu.async_copy(targ_hbm.at[pl.ds(off, _CHUNK)], t_v, sem_t)
    cp_b = pltpu.async_copy(batch_hbm.at[pl.ds(off, _CHUNK)], b_v, sem_b)
    cp_p.wait()
    cp_t.wait()
    cp_b.wait()

    zeros = jnp.zeros((_LANE,), jnp.float32)
    lane = jax.lax.iota(jnp.int32, _LANE)
    acc_mask = lane < 2
    acc_off = jnp.where(lane == 1, jnp.int32(128), jnp.int32(0))

    @pl.loop(0, 256 // _LANE)
    def _(j):
        bins_v[pl.ds(j * _LANE, _LANE)] = zeros

    def acc(g, s, c):
        idx = jnp.full((_LANE,), g, jnp.int32) + acc_off
        val = jnp.where(lane == 0, s, c)
        plsc.addupdate_scatter(bins_v, [idx], val, mask=acc_mask)

    def vec_rel(base):
        p16 = p_v[pl.ds(base, _LANE)]
        t16 = t_v[pl.ds(base, _LANE)]
        return jnp.abs(p16 - t16) / (jnp.abs(t16) + _EPS)

    def vec_slow(base):
        b16 = b_v[pl.ds(base, _LANE)]
        rel = vec_rel(base)
        b0 = b16[0]
        b15 = b16[_LANE - 1]

        def gbody(g, carry):
            m = b16 == g
            s = jnp.sum(jnp.where(m, rel, jnp.float32(0.0)))
            c = jnp.sum(jnp.where(m, jnp.float32(1.0), jnp.float32(0.0)))
            acc(g, s, c)
            return carry

        jax.lax.fori_loop(b0, b15 + 1, gbody, jnp.int32(0))

    @plsc.parallel_loop(0, _CHUNK // (4 * _LANE), unroll=4)
    def _(i):
        base = i * (4 * _LANE)
        b_head = b_v[pl.ds(base, _LANE)]
        b_tail = b_v[pl.ds(base + 3 * _LANE, _LANE)]
        b_first = b_head[0]
        b_last = b_tail[_LANE - 1]

        @pl.when(b_first == b_last)
        def _():
            r = (
                (vec_rel(base) + vec_rel(base + _LANE))
                + (vec_rel(base + 2 * _LANE) + vec_rel(base + 3 * _LANE))
            )
            acc(b_first, jnp.sum(r), jnp.float32(4 * _LANE))

        @pl.when(b_first != b_last)
        def _():
            for k in range(4):
                vec_slow(base + k * _LANE)

    pltpu.sync_copy(bins_v, out_hbm.at[chunk])


def _sc_partials(pred, target, batch):
    mesh = plsc.VectorSubcoreMesh(core_axis_name="c", subcore_axis_name="s")
    cp = pltpu.CompilerParams()
    if "needs_layout_passes" in pltpu.CompilerParams.__dataclass_fields__:
        cp = dataclasses.replace(cp, needs_layout_passes=False)
    kern = pl.kernel(
        _sc_body,
        out_type=jax.ShapeDtypeStruct((_NSC, 256), jnp.float32),
        mesh=mesh,
        scratch_types=[
            pltpu.VMEM((_CHUNK,), jnp.float32),
            pltpu.VMEM((_CHUNK,), jnp.float32),
            pltpu.VMEM((_CHUNK,), jnp.int32),
            pltpu.VMEM((256,), jnp.float32),
            pltpu.SemaphoreType.DMA,
            pltpu.SemaphoreType.DMA,
            pltpu.SemaphoreType.DMA,
        ],
        compiler_params=cp,
    )
    return kern(pred, target, batch)


def _finalize_body(part_ref, last_ref, out_ref):
    p = part_ref[...]
    red = jnp.sum(p, axis=0)
    num_graphs = last_ref[0, 0] + 1
    ids = jax.lax.broadcasted_iota(jnp.int32, (1, 128), 1)
    sums = red[0:1, :]
    cnts = red[1:2, :]
    means = sums / cnts
    valid = (ids < num_graphs) & (ids < _NUM_GRAPHS)
    total = jnp.sum(jnp.where(valid, means, 0.0))
    result = total / num_graphs.astype(jnp.float32) * 10000.0
    out_ref[...] = jnp.broadcast_to(result, (1, 1))


def kernel(pred, target, batch, x):
    del x
    n = pred.shape[0]
    batch = batch.astype(jnp.int32)
    padded = _NSC * _CHUNK
    pad = padded - n
    pred2 = jnp.pad(pred, (0, pad))
    targ2 = jnp.pad(target, (0, pad))
    batch2 = jnp.pad(batch, (0, pad), constant_values=_SENTINEL)
    partials = _sc_partials(pred2, targ2, batch2).reshape(_NSC, 2, 128)
    last = batch[n - 1 :].reshape(1, 1)
    out = pl.pallas_call(
        _finalize_body,
        out_shape=jax.ShapeDtypeStruct((1, 1), jnp.float32),
    )(partials, last)
    return out.reshape(())


# speedup vs baseline: 1.1100x; 1.1100x over previous
import dataclasses
import functools

import jax
import jax.numpy as jnp
from jax.experimental import pallas as pl
from jax.experimental.pallas import tpu as pltpu
from jax.experimental.pallas import tpu_sc as plsc

_EPS = 0.1
_NUM_GRAPHS = 64
_LANE = 16
_NSC = 16
_CHUNK = 6272
_SENTINEL = 64


def _sc_body(
    n,
    pred_hbm,
    targ_hbm,
    batch_hbm,
    out_hbm,
    p_v,
    t_v,
    b_v,
    bins_v,
    shared_v,
    agg_v,
    tail_v,
    res_v,
    sem_p,
    sem_t,
    sem_b,
):
    sid = jax.lax.axis_index("s")
    off = sid * _CHUNK
    cp_p = pltpu.async_copy(pred_hbm.at[pl.ds(off, _CHUNK)], p_v, sem_p)
    cp_t = pltpu.async_copy(targ_hbm.at[pl.ds(off, _CHUNK)], t_v, sem_t)
    cp_b = pltpu.async_copy(batch_hbm.at[pl.ds(off, _CHUNK)], b_v, sem_b)
    cp_p.wait()
    cp_t.wait()
    cp_b.wait()

    zeros = jnp.zeros((_LANE,), jnp.float32)
    lane = jax.lax.iota(jnp.int32, _LANE)
    acc_mask = lane < 2
    acc_off = jnp.where(lane == 1, jnp.int32(128), jnp.int32(0))

    @pl.loop(0, 256 // _LANE)
    def _(j):
        bins_v[pl.ds(j * _LANE, _LANE)] = zeros

    def acc(g, s, c):
        idx = jnp.full((_LANE,), g, jnp.int32) + acc_off
        val = jnp.where(lane == 0, s, c)
        plsc.addupdate_scatter(bins_v, [idx], val, mask=acc_mask)

    def vec_rel(base):
        p16 = p_v[pl.ds(base, _LANE)]
        t16 = t_v[pl.ds(base, _LANE)]
        return jnp.abs(p16 - t16) / (jnp.abs(t16) + _EPS)

    def vec_slow(base):
        b16 = b_v[pl.ds(base, _LANE)]
        rel = vec_rel(base)
        b0 = b16[0]
        b15 = b16[_LANE - 1]

        def gbody(g, carry):
            m = b16 == g
            s = jnp.sum(jnp.where(m, rel, jnp.float32(0.0)))
            c = jnp.sum(jnp.where(m, jnp.float32(1.0), jnp.float32(0.0)))
            acc(g, s, c)
            return carry

        jax.lax.fori_loop(b0, b15 + 1, gbody, jnp.int32(0))

    @pl.loop(0, _CHUNK // (4 * _LANE))
    def _(i):
        base = i * (4 * _LANE)
        b_head = b_v[pl.ds(base, _LANE)]
        b_tail = b_v[pl.ds(base + 3 * _LANE, _LANE)]
        b_first = b_head[0]
        b_last = b_tail[_LANE - 1]

        @pl.when(b_first == b_last)
        def _():
            r = (
                (vec_rel(base) + vec_rel(base + _LANE))
                + (vec_rel(base + 2 * _LANE) + vec_rel(base + 3 * _LANE))
            )
            acc(b_first, jnp.sum(r), jnp.float32(4 * _LANE))

        @pl.when(b_first != b_last)
        def _():
            for k in range(4):
                vec_slow(base + k * _LANE)

    pltpu.sync_copy(bins_v, shared_v.at[sid])
    plsc.subcore_barrier()

    @pl.when(sid == 0)
    def _():
        pltpu.sync_copy(shared_v, agg_v)
        pltpu.sync_copy(batch_hbm.at[pl.ds(n - _LANE, _LANE)], tail_v)
        ng = tail_v[...][_LANE - 1] + 1

        tv = jnp.zeros((_LANE,), jnp.float32)
        for j in range(_NUM_GRAPHS // _LANE):

            def red(col, accv):
                def rbody(part, a):
                    return a + agg_v[part, pl.ds(col, _LANE)]

                return jax.lax.fori_loop(0, _NSC, rbody, accv)

            s16 = red(j * _LANE, jnp.zeros((_LANE,), jnp.float32))
            c16 = red(128 + j * _LANE, jnp.zeros((_LANE,), jnp.float32))
            ids = lane + j * _LANE
            tv = tv + jnp.where(ids < ng, s16 / c16, jnp.float32(0.0))

        total = jnp.sum(tv)
        ngf = ng.astype(jnp.float32)
        res_v[...] = (
            jnp.full((_LANE,), total, jnp.float32)
            / jnp.full((_LANE,), ngf, jnp.float32)
            * 10000.0
        )
        pltpu.sync_copy(res_v, out_hbm)


def _sc_call(pred, target, batch, n):
    mesh = plsc.VectorSubcoreMesh(
        core_axis_name="c", subcore_axis_name="s", num_cores=1
    )
    cp = pltpu.CompilerParams()
    if "needs_layout_passes" in pltpu.CompilerParams.__dataclass_fields__:
        cp = dataclasses.replace(cp, needs_layout_passes=False)
    kern = pl.kernel(
        functools.partial(_sc_body, n),
        out_type=jax.ShapeDtypeStruct((_LANE,), jnp.float32),
        mesh=mesh,
        scratch_types=[
            pltpu.VMEM((_CHUNK,), jnp.float32),
            pltpu.VMEM((_CHUNK,), jnp.float32),
            pltpu.VMEM((_CHUNK,), jnp.int32),
            pltpu.VMEM((256,), jnp.float32),
            pltpu.VMEM_SHARED((_NSC, 256), jnp.float32),
            pltpu.VMEM((_NSC, 256), jnp.float32),
            pltpu.VMEM((_LANE,), jnp.int32),
            pltpu.VMEM((_LANE,), jnp.float32),
            pltpu.SemaphoreType.DMA,
            pltpu.SemaphoreType.DMA,
            pltpu.SemaphoreType.DMA,
        ],
        compiler_params=cp,
    )
    return kern(pred, target, batch)


def kernel(pred, target, batch, x):
    del x
    n = pred.shape[0]
    batch = batch.astype(jnp.int32)
    padded = _NSC * _CHUNK
    pad = padded - n
    pred2 = jnp.pad(pred, (0, pad))
    targ2 = jnp.pad(target, (0, pad))
    batch2 = jnp.pad(batch, (0, pad), constant_values=_SENTINEL)
    out = _sc_call(pred2, targ2, batch2, n)
    return out[0].reshape(())


# speedup vs baseline: 1.1969x; 1.0784x over previous
import dataclasses
import functools

import jax
import jax.numpy as jnp
from jax.experimental import pallas as pl
from jax.experimental.pallas import tpu as pltpu
from jax.experimental.pallas import tpu_sc as plsc

_EPS = 0.1
_NUM_GRAPHS = 64
_LANE = 16
_NSC = 16
_CHUNK = 6272
_SENTINEL = 64


def _sc_body(
    n,
    pred_hbm,
    targ_hbm,
    batch_hbm,
    out_hbm,
    p_v,
    t_v,
    b_v,
    bins_v,
    shared_v,
    agg_v,
    tail_v,
    res_v,
    sem_p,
    sem_t,
    sem_b,
):
    sid = jax.lax.axis_index("s")
    off = sid * _CHUNK
    cp_p = pltpu.async_copy(pred_hbm.at[pl.ds(off, _CHUNK)], p_v, sem_p)
    cp_t = pltpu.async_copy(targ_hbm.at[pl.ds(off, _CHUNK)], t_v, sem_t)
    cp_b = pltpu.async_copy(batch_hbm.at[pl.ds(off, _CHUNK)], b_v, sem_b)
    cp_p.wait()
    cp_t.wait()
    cp_b.wait()

    zeros = jnp.zeros((_LANE,), jnp.float32)
    lane = jax.lax.iota(jnp.int32, _LANE)
    acc_mask = lane < 2
    acc_off = jnp.where(lane == 1, jnp.int32(128), jnp.int32(0))

    @pl.loop(0, 256 // _LANE)
    def _(j):
        bins_v[pl.ds(j * _LANE, _LANE)] = zeros

    def acc(g, s, c):
        idx = jnp.full((_LANE,), g, jnp.int32) + acc_off
        val = jnp.where(lane == 0, s, c)
        plsc.addupdate_scatter(bins_v, [idx], val, mask=acc_mask)

    def vec_rel(base):
        p16 = p_v[pl.ds(base, _LANE)]
        t16 = t_v[pl.ds(base, _LANE)]
        return jnp.abs(p16 - t16) / (jnp.abs(t16) + _EPS)

    def vec_slow(base):
        b16 = b_v[pl.ds(base, _LANE)]
        rel = vec_rel(base)
        b0 = b16[0]
        b15 = b16[_LANE - 1]

        def gbody(g, carry):
            m = b16 == g
            s = jnp.sum(jnp.where(m, rel, jnp.float32(0.0)))
            c = jnp.sum(jnp.where(m, jnp.float32(1.0), jnp.float32(0.0)))
            acc(g, s, c)
            return carry

        jax.lax.fori_loop(b0, b15 + 1, gbody, jnp.int32(0))

    def wbody(i, carry):
        g, accv, c = carry
        base = i * (4 * _LANE)
        b_first = b_v[pl.ds(base, _LANE)][0]
        b_last = b_v[pl.ds(base + 3 * _LANE, _LANE)][_LANE - 1]
        r = (vec_rel(base) + vec_rel(base + _LANE)) + (
            vec_rel(base + 2 * _LANE) + vec_rel(base + 3 * _LANE)
        )
        uniform = b_first == b_last
        same = jnp.logical_and(uniform, b_first == g)

        def fast_append(_):
            return (g, accv + r, c + jnp.float32(4 * _LANE))

        def changed(_):
            acc(g, jnp.sum(accv), c)

            def uni(_):
                return (b_first, r, jnp.float32(4 * _LANE))

            def slow(_):
                for k in range(4):
                    vec_slow(base + k * _LANE)
                return (b_last, jnp.zeros((_LANE,), jnp.float32), jnp.float32(0.0))

            return jax.lax.cond(uniform, uni, slow, 0)

        return jax.lax.cond(same, fast_append, changed, 0)

    g0 = b_v[pl.ds(0, _LANE)][0]
    carry0 = (g0, jnp.zeros((_LANE,), jnp.float32), jnp.float32(0.0))
    gf, accf, cf = jax.lax.fori_loop(0, _CHUNK // (4 * _LANE), wbody, carry0)
    acc(gf, jnp.sum(accf), cf)

    pltpu.sync_copy(bins_v, shared_v.at[sid])
    plsc.subcore_barrier()

    @pl.when(sid == 0)
    def _():
        pltpu.sync_copy(shared_v, agg_v)
        pltpu.sync_copy(batch_hbm.at[pl.ds(n - _LANE, _LANE)], tail_v)
        ng = tail_v[...][_LANE - 1] + 1

        tv = jnp.zeros((_LANE,), jnp.float32)
        for j in range(_NUM_GRAPHS // _LANE):

            def red(col, accv):
                def rbody(part, a):
                    return a + agg_v[part, pl.ds(col, _LANE)]

                return jax.lax.fori_loop(0, _NSC, rbody, accv)

            s16 = red(j * _LANE, jnp.zeros((_LANE,), jnp.float32))
            c16 = red(128 + j * _LANE, jnp.zeros((_LANE,), jnp.float32))
            ids = lane + j * _LANE
            tv = tv + jnp.where(ids < ng, s16 / c16, jnp.float32(0.0))

        total = jnp.sum(tv)
        ngf = ng.astype(jnp.float32)
        res_v[...] = (
            jnp.full((_LANE,), total, jnp.float32)
            / jnp.full((_LANE,), ngf, jnp.float32)
            * 10000.0
        )
        pltpu.sync_copy(res_v, out_hbm)


def _sc_call(pred, target, batch, n):
    mesh = plsc.VectorSubcoreMesh(
        core_axis_name="c", subcore_axis_name="s", num_cores=1
    )
    cp = pltpu.CompilerParams()
    if "needs_layout_passes" in pltpu.CompilerParams.__dataclass_fields__:
        cp = dataclasses.replace(cp, needs_layout_passes=False)
    kern = pl.kernel(
        functools.partial(_sc_body, n),
        out_type=jax.ShapeDtypeStruct((_LANE,), jnp.float32),
        mesh=mesh,
        scratch_types=[
            pltpu.VMEM((_CHUNK,), jnp.float32),
            pltpu.VMEM((_CHUNK,), jnp.float32),
            pltpu.VMEM((_CHUNK,), jnp.int32),
            pltpu.VMEM((256,), jnp.float32),
            pltpu.VMEM_SHARED((_NSC, 256), jnp.float32),
            pltpu.VMEM((_NSC, 256), jnp.float32),
            pltpu.VMEM((_LANE,), jnp.int32),
            pltpu.VMEM((_LANE,), jnp.float32),
            pltpu.SemaphoreType.DMA,
            pltpu.SemaphoreType.DMA,
            pltpu.SemaphoreType.DMA,
        ],
        compiler_params=cp,
    )
    return kern(pred, target, batch)


def kernel(pred, target, batch, x):
    del x
    n = pred.shape[0]
    batch = batch.astype(jnp.int32)
    padded = _NSC * _CHUNK
    pad = padded - n
    pred2 = jnp.pad(pred, (0, pad))
    targ2 = jnp.pad(target, (0, pad))
    batch2 = jnp.pad(batch, (0, pad), constant_values=_SENTINEL)
    out = _sc_call(pred2, targ2, batch2, n)
    return out[0].reshape(())
